# baseline (speedup 1.0000x reference)
import functools

import jax
import jax.numpy as jnp
from jax import lax
from jax.experimental import pallas as pl
from jax.experimental.pallas import tpu as pltpu
from jax.experimental.pallas import tpu_sc as plsc

_BN = 1024
_MERGE = 4
_NIMG = 8
_L = 16
_NS = 16
_CHUNK = 512


def _fold(v, iot):
    for d in (1, 2, 4, 8):
        v = v + v.at[(iot + d) & (_L - 1)].get(mode="promise_in_bounds")
    return v


def _prefix(v, iot):
    for d in (1, 2, 4, 8):
        gate = jnp.minimum(jnp.maximum(iot - (d - 1), 0), 1)
        sh = v.at[jnp.maximum(iot - d, 0)].get(mode="promise_in_bounds")
        v = v + sh * gate
    return v


def _clamp01(v):
    return jnp.minimum(jnp.maximum(v, 0), 1)


def _sc_bounds_body(mask_hbm, g3_hbm, out_hbm, mask_v, g3_v, obuf):
    cid = lax.axis_index("c")
    sid = lax.axis_index("s")

    @pl.when(sid == 0)
    def _():
        row = _NS * _CHUNK
        pltpu.sync_copy(mask_hbm.at[pl.ds(cid * row, row)], mask_v)
        pltpu.sync_copy(g3_hbm, g3_v)
        iot = lax.iota(jnp.int32, _L)

        ntv = g3_v[0] * g3_v[1] * g3_v[2]
        cums = []
        c = jnp.int32(0)
        for i in range(_NIMG):
            c = c + lax.div(ntv[i], jnp.int32(_MERGE))
            cums.append(c)

        def step(k, carry):
            run = carry[0]
            parts = carry[1:]
            mvec = mask_v[pl.ds(k * _L, _L)]
            pf = _prefix(mvec, iot)
            cnt = pf + run
            new_parts = tuple(
                p + _clamp01((jnp.zeros((_L,), jnp.int32) + cums[j]) - cnt + 1)
                for j, p in enumerate(parts))
            return (run + pf[_L - 1],) + new_parts

        init = (jnp.int32(0),) + tuple(
            jnp.zeros((_L,), jnp.int32) for _ in range(_NIMG))
        res = lax.fori_loop(0, row // _L, step, init)
        bvec = jnp.zeros((_L,), jnp.int32)
        for j in range(_NIMG):
            lane_j = 1 - jnp.minimum(jnp.abs(iot - j), 1)
            bvec = bvec + _fold(res[1 + j], iot) * lane_j
        obuf[0, :] = bvec
        pltpu.sync_copy(obuf, out_hbm.at[pl.ds(cid, 1)])


def _sc_bounds(mask_flat, g3, b):
    mesh = plsc.VectorSubcoreMesh(core_axis_name="c", subcore_axis_name="s")
    f = functools.partial(
        pl.kernel,
        mesh=mesh,
        out_type=jax.ShapeDtypeStruct((b, _L), jnp.int32),
        scratch_types=[
            pltpu.VMEM((_NS * _CHUNK,), jnp.int32),
            pltpu.VMEM((3, _L), jnp.int32),
            pltpu.VMEM((1, _L), jnp.int32),
        ],
    )(_sc_bounds_body)
    return f(mask_flat, g3)


def _tc_body(nc_ref, bounds_ref, mask_ref, feat_ref, table_ref, out_ref):
    bi = pl.program_id(0)
    j = pl.program_id(1)

    m = mask_ref[0, 0]
    bn = m.shape[0]
    pos = j * bn + lax.broadcasted_iota(jnp.int32, (bn, 1), 0)

    nc = nc_ref[0]
    ncs = jnp.maximum(nc, 1)
    k_iota = lax.broadcasted_iota(jnp.int32, (1, _NIMG), 1)

    prev = (m > 0) & (nc > 1)
    onehot = jnp.zeros((bn, _NIMG), jnp.bfloat16)
    for i in range(_NIMG):
        bj = bounds_ref[bi, i]
        lt = pos < bj
        ind = prev & lt
        sel = k_iota == (jnp.int32(i) % ncs)
        onehot = onehot + (ind & sel).astype(jnp.bfloat16)
        prev = prev & jnp.logical_not(lt)

    emb = lax.dot_general(
        onehot, table_ref[...], (((1,), (0,)), ((), ())),
        preferred_element_type=jnp.float32,
    )
    out_ref[0] = feat_ref[0] + emb


def _tc_run(features, mask_i32, bounds, nc_arr, table_bf16):
    b, n, d = features.shape
    nb = n // _BN
    mask4 = mask_i32.reshape(b, nb, _BN, 1)
    return pl.pallas_call(
        _tc_body,
        grid=(b, nb),
        in_specs=[
            pl.BlockSpec(memory_space=pltpu.SMEM),
            pl.BlockSpec(memory_space=pltpu.SMEM),
            pl.BlockSpec((1, 1, _BN, 1), lambda b_, j: (b_, j, 0, 0)),
            pl.BlockSpec((1, _BN, d), lambda b_, j: (b_, j, 0)),
            pl.BlockSpec((_NIMG, d), lambda b_, j: (0, 0)),
        ],
        out_specs=pl.BlockSpec((1, _BN, d), lambda b_, j: (b_, j, 0)),
        out_shape=jax.ShapeDtypeStruct((b, n, d), features.dtype),
    )(nc_arr, bounds, mask4, features, table_bf16)


def kernel(features, vision_mask, image_grid_thw, num_cameras, camera_table):
    nc_arr = jnp.asarray(num_cameras, jnp.int32).reshape(1)
    grid_i32 = jnp.asarray(image_grid_thw, jnp.int32)
    mask_i32 = jnp.asarray(vision_mask, jnp.int32)
    table_bf16 = camera_table.astype(jnp.bfloat16)
    g3 = jnp.zeros((3, _L), jnp.int32).at[:, :_NIMG].set(grid_i32.T)
    bounds = _sc_bounds(mask_i32.reshape(-1), g3, features.shape[0])
    return _tc_run(features, mask_i32, bounds, nc_arr, table_bf16)


# speedup vs baseline: 1.0132x; 1.0132x over previous
import functools

import jax
import jax.numpy as jnp
from jax import lax
from jax.experimental import pallas as pl
from jax.experimental.pallas import tpu as pltpu
from jax.experimental.pallas import tpu_sc as plsc

_BN = 1024
_MERGE = 4
_NIMG = 8
_L = 16
_NS = 16
_CHUNK = 512


def _fold(v, iot):
    for d in (1, 2, 4, 8):
        v = v + v.at[(iot + d) & (_L - 1)].get(mode="promise_in_bounds")
    return v


def _prefix(v, iot):
    for d in (1, 2, 4, 8):
        gate = jnp.minimum(jnp.maximum(iot - (d - 1), 0), 1)
        sh = v.at[jnp.maximum(iot - d, 0)].get(mode="promise_in_bounds")
        v = v + sh * gate
    return v


def _clamp01(v):
    return jnp.minimum(jnp.maximum(v, 0), 1)


def _sc_bounds_body(mask_hbm, g3_hbm, out_hbm, mask_v, g3_v, obuf):
    cid = lax.axis_index("c")
    sid = lax.axis_index("s")

    @pl.when(sid == 0)
    def _():
        row = _NS * _CHUNK
        pltpu.sync_copy(mask_hbm.at[pl.ds(cid * row, row)], mask_v)
        pltpu.sync_copy(g3_hbm, g3_v)
        iot = lax.iota(jnp.int32, _L)

        ntv = g3_v[0] * g3_v[1] * g3_v[2]
        cums = []
        c = jnp.int32(0)
        for i in range(_NIMG):
            c = c + lax.div(ntv[i], jnp.int32(_MERGE))
            cums.append(c)

        def step(k, carry):
            run = carry[0]
            parts = carry[1:]
            mvec = mask_v[pl.ds(k * _L, _L)]
            pf = _prefix(mvec, iot)
            cnt = pf + run
            new_parts = tuple(
                p + _clamp01((jnp.zeros((_L,), jnp.int32) + cums[j]) - cnt + 1)
                for j, p in enumerate(parts))
            return (run + pf[_L - 1],) + new_parts

        init = (jnp.int32(0),) + tuple(
            jnp.zeros((_L,), jnp.int32) for _ in range(_NIMG))
        res = lax.fori_loop(0, row // _L, step, init)
        bvec = jnp.zeros((_L,), jnp.int32)
        for j in range(_NIMG):
            lane_j = 1 - jnp.minimum(jnp.abs(iot - j), 1)
            bvec = bvec + _fold(res[1 + j], iot) * lane_j
        obuf[0, :] = bvec
        pltpu.sync_copy(obuf, out_hbm.at[pl.ds(cid, 1)])


def _sc_bounds(mask_flat, g3, b):
    mesh = plsc.VectorSubcoreMesh(core_axis_name="c", subcore_axis_name="s")
    f = functools.partial(
        pl.kernel,
        mesh=mesh,
        out_type=jax.ShapeDtypeStruct((b, _L), jnp.int32),
        scratch_types=[
            pltpu.VMEM((_NS * _CHUNK,), jnp.int32),
            pltpu.VMEM((3, _L), jnp.int32),
            pltpu.VMEM((1, _L), jnp.int32),
        ],
    )(_sc_bounds_body)
    return f(mask_flat, g3)


def _tc_body(nc_ref, bounds_ref, mask_ref, feat_ref, table_ref, out_ref):
    bi = pl.program_id(0)
    j = pl.program_id(1)

    m = mask_ref[0, 0]
    bn = m.shape[0]
    pos = j * bn + lax.broadcasted_iota(jnp.int32, (bn, 1), 0)

    nc = nc_ref[0]
    ncs = jnp.maximum(nc, 1)

    pos0 = j * bn
    i_star = jnp.int32(0)
    for t in range(_NIMG):
        i_star = i_star + (bounds_ref[bi, t] <= pos0).astype(jnp.int32)
    nxt = bounds_ref[bi, jnp.minimum(i_star, _NIMG - 1)]
    uniform = (i_star == _NIMG) | (nxt >= pos0 + bn)
    cam_star = jnp.int32(0)
    for t in range(_NIMG):
        cam_star = cam_star + (i_star == t).astype(jnp.int32) * (
            jnp.int32(t) % ncs)
    gate = ((i_star < _NIMG) & (nc > 1)).astype(jnp.float32)

    @pl.when(uniform)
    def _():
        trow = jnp.zeros((1, table_ref.shape[1]), jnp.float32)
        for t in range(_NIMG):
            g_t = (cam_star == t).astype(jnp.float32)
            trow = trow + table_ref[t:t + 1, :].astype(jnp.float32) * g_t
        out_ref[0] = feat_ref[0] + (m.astype(jnp.float32) * gate) * trow

    @pl.when(jnp.logical_not(uniform))
    def _():
        k_iota = lax.broadcasted_iota(jnp.int32, (1, _NIMG), 1)
        prev = (m > 0) & (nc > 1)
        onehot = jnp.zeros((bn, _NIMG), jnp.bfloat16)
        for i in range(_NIMG):
            bj = bounds_ref[bi, i]
            lt = pos < bj
            ind = prev & lt
            sel = k_iota == (jnp.int32(i) % ncs)
            onehot = onehot + (ind & sel).astype(jnp.bfloat16)
            prev = prev & jnp.logical_not(lt)
        emb = lax.dot_general(
            onehot, table_ref[...], (((1,), (0,)), ((), ())),
            preferred_element_type=jnp.float32,
        )
        out_ref[0] = feat_ref[0] + emb


def _tc_run(features, mask_i32, bounds, nc_arr, table_bf16):
    b, n, d = features.shape
    nb = n // _BN
    mask4 = mask_i32.reshape(b, nb, _BN, 1)
    return pl.pallas_call(
        _tc_body,
        grid=(b, nb),
        in_specs=[
            pl.BlockSpec(memory_space=pltpu.SMEM),
            pl.BlockSpec(memory_space=pltpu.SMEM),
            pl.BlockSpec((1, 1, _BN, 1), lambda b_, j: (b_, j, 0, 0)),
            pl.BlockSpec((1, _BN, d), lambda b_, j: (b_, j, 0)),
            pl.BlockSpec((_NIMG, d), lambda b_, j: (0, 0)),
        ],
        out_specs=pl.BlockSpec((1, _BN, d), lambda b_, j: (b_, j, 0)),
        out_shape=jax.ShapeDtypeStruct((b, n, d), features.dtype),
    )(nc_arr, bounds, mask4, features, table_bf16)


def kernel(features, vision_mask, image_grid_thw, num_cameras, camera_table):
    nc_arr = jnp.asarray(num_cameras, jnp.int32).reshape(1)
    grid_i32 = jnp.asarray(image_grid_thw, jnp.int32)
    mask_i32 = jnp.asarray(vision_mask, jnp.int32)
    table_bf16 = camera_table.astype(jnp.bfloat16)
    g3 = jnp.zeros((3, _L), jnp.int32).at[:, :_NIMG].set(grid_i32.T)
    bounds = _sc_bounds(mask_i32.reshape(-1), g3, features.shape[0])
    return _tc_run(features, mask_i32, bounds, nc_arr, table_bf16)


# speedup vs baseline: 1.0147x; 1.0015x over previous
import functools

import jax
import jax.numpy as jnp
from jax import lax
from jax.experimental import pallas as pl
from jax.experimental.pallas import tpu as pltpu
from jax.experimental.pallas import tpu_sc as plsc

_BN = 1024
_MERGE = 4
_NIMG = 8
_L = 16
_NS = 16
_CHUNK = 512


def _fold(v, iot):
    for d in (1, 2, 4, 8):
        v = v + v.at[(iot + d) & (_L - 1)].get(mode="promise_in_bounds")
    return v


def _prefix(v, iot):
    for d in (1, 2, 4, 8):
        gate = jnp.minimum(jnp.maximum(iot - (d - 1), 0), 1)
        sh = v.at[jnp.maximum(iot - d, 0)].get(mode="promise_in_bounds")
        v = v + sh * gate
    return v


def _clamp01(v):
    return jnp.minimum(jnp.maximum(v, 0), 1)


def _sc_bounds_body(mask_hbm, g3_hbm, out_hbm, mask_v, g3_v, obuf):
    cid = lax.axis_index("c")
    sid = lax.axis_index("s")

    @pl.when(sid == 0)
    def _():
        row = _NS * _CHUNK
        pltpu.sync_copy(mask_hbm.at[pl.ds(cid * row, row)], mask_v)
        pltpu.sync_copy(g3_hbm, g3_v)
        iot = lax.iota(jnp.int32, _L)

        ntv = g3_v[0] * g3_v[1] * g3_v[2]
        cums = []
        c = jnp.int32(0)
        for i in range(_NIMG):
            c = c + lax.div(ntv[i], jnp.int32(_MERGE))
            cums.append(c)

        def step(k, carry):
            run = carry[0]
            parts = carry[1:]
            mvec = mask_v[pl.ds(k * _L, _L)]
            pf = _prefix(mvec, iot)
            cnt = pf + run
            new_parts = tuple(
                p + _clamp01((jnp.zeros((_L,), jnp.int32) + cums[j]) - cnt + 1)
                for j, p in enumerate(parts))
            return (run + pf[_L - 1],) + new_parts

        init = (jnp.int32(0),) + tuple(
            jnp.zeros((_L,), jnp.int32) for _ in range(_NIMG))
        res = lax.fori_loop(0, row // _L, step, init)
        bvec = jnp.zeros((_L,), jnp.int32)
        for j in range(_NIMG):
            lane_j = 1 - jnp.minimum(jnp.abs(iot - j), 1)
            bvec = bvec + _fold(res[1 + j], iot) * lane_j
        obuf[0, :] = bvec
        pltpu.sync_copy(obuf, out_hbm.at[pl.ds(cid, 1)])


def _sc_bounds(mask_flat, g3, b):
    mesh = plsc.VectorSubcoreMesh(core_axis_name="c", subcore_axis_name="s")
    f = functools.partial(
        pl.kernel,
        mesh=mesh,
        out_type=jax.ShapeDtypeStruct((b, _L), jnp.int32),
        scratch_types=[
            pltpu.VMEM((_NS * _CHUNK,), jnp.int32),
            pltpu.VMEM((3, _L), jnp.int32),
            pltpu.VMEM((1, _L), jnp.int32),
        ],
    )(_sc_bounds_body)
    return f(mask_flat, g3)


def _tc_body(nc_ref, bounds_ref, mask_ref, feat_ref, table_ref, out_ref):
    bi = pl.program_id(0)
    j = pl.program_id(1)

    m = mask_ref[bi, j]
    bn = m.shape[0]
    pos = j * bn + lax.broadcasted_iota(jnp.int32, (bn, 1), 0)

    nc = nc_ref[0]
    ncs = jnp.maximum(nc, 1)

    pos0 = j * bn
    i_star = jnp.int32(0)
    for t in range(_NIMG):
        i_star = i_star + (bounds_ref[bi, t] <= pos0).astype(jnp.int32)
    nxt = bounds_ref[bi, jnp.minimum(i_star, _NIMG - 1)]
    uniform = (i_star == _NIMG) | (nxt >= pos0 + bn)
    cam_star = jnp.int32(0)
    for t in range(_NIMG):
        cam_star = cam_star + (i_star == t).astype(jnp.int32) * (
            jnp.int32(t) % ncs)
    gate = ((i_star < _NIMG) & (nc > 1)).astype(jnp.float32)

    @pl.when(uniform)
    def _():
        trow = jnp.zeros((1, table_ref.shape[1]), jnp.float32)
        for t in range(_NIMG):
            g_t = (cam_star == t).astype(jnp.float32)
            trow = trow + table_ref[t:t + 1, :].astype(jnp.float32) * g_t
        out_ref[0] = feat_ref[0] + (m.astype(jnp.float32) * gate) * trow

    @pl.when(jnp.logical_not(uniform))
    def _():
        k_iota = lax.broadcasted_iota(jnp.int32, (1, _NIMG), 1)
        prev = (m > 0) & (nc > 1)
        onehot = jnp.zeros((bn, _NIMG), jnp.bfloat16)
        for i in range(_NIMG):
            bj = bounds_ref[bi, i]
            lt = pos < bj
            ind = prev & lt
            sel = k_iota == (jnp.int32(i) % ncs)
            onehot = onehot + (ind & sel).astype(jnp.bfloat16)
            prev = prev & jnp.logical_not(lt)
        emb = lax.dot_general(
            onehot, table_ref[...], (((1,), (0,)), ((), ())),
            preferred_element_type=jnp.float32,
        )
        out_ref[0] = feat_ref[0] + emb


def _tc_run(features, mask_i32, bounds, nc_arr, table_bf16):
    b, n, d = features.shape
    nb = n // _BN
    mask4 = mask_i32.reshape(b, nb, _BN, 1)
    return pl.pallas_call(
        _tc_body,
        grid=(b, nb),
        in_specs=[
            pl.BlockSpec(memory_space=pltpu.SMEM),
            pl.BlockSpec(memory_space=pltpu.SMEM),
            pl.BlockSpec((b, nb, _BN, 1), lambda b_, j: (0, 0, 0, 0)),
            pl.BlockSpec((1, _BN, d), lambda b_, j: (b_, j, 0)),
            pl.BlockSpec((_NIMG, d), lambda b_, j: (0, 0)),
        ],
        out_specs=pl.BlockSpec((1, _BN, d), lambda b_, j: (b_, j, 0)),
        out_shape=jax.ShapeDtypeStruct((b, n, d), features.dtype),
    )(nc_arr, bounds, mask4, features, table_bf16)


def kernel(features, vision_mask, image_grid_thw, num_cameras, camera_table):
    nc_arr = jnp.asarray(num_cameras, jnp.int32).reshape(1)
    grid_i32 = jnp.asarray(image_grid_thw, jnp.int32)
    mask_i32 = jnp.asarray(vision_mask, jnp.int32)
    table_bf16 = camera_table.astype(jnp.bfloat16)
    g3 = jnp.zeros((3, _L), jnp.int32).at[:, :_NIMG].set(grid_i32.T)
    bounds = _sc_bounds(mask_i32.reshape(-1), g3, features.shape[0])
    return _tc_run(features, mask_i32, bounds, nc_arr, table_bf16)
